# Initial kernel scaffold; baseline (speedup 1.0000x reference)
import jax
import jax.numpy as jnp
from jax.experimental import pallas as pl


def kernel(x, edge_index, W1a, b1a, g1, be1, W1b, b1b, W2a, b2a, g2, be2, W2b, b2b):
    raise NotImplementedError("write your pallas kernel here")


# speedup vs baseline: 4.2492x; 4.2492x over previous
import functools

import jax
import jax.numpy as jnp
from jax import lax
from jax.experimental import pallas as pl
from jax.experimental.pallas import tpu as pltpu
from jax.experimental.pallas import tpu_sc as plsc

N = 10000
E = 320000
D = 128
BN_EPS = 1e-5

NC = 2
NS = 16
NW = NC * NS
CHUNK = 128
CPW = 79
EPW = CPW * CHUNK
E_PAD = NW * EPW
N_PAD = 10240
RPT = N_PAD // NS


def _sc_aggregate(h, src3, dst3):
    mesh = plsc.VectorSubcoreMesh(core_axis_name="c", subcore_axis_name="s")

    @functools.partial(
        pl.kernel,
        out_type=jax.ShapeDtypeStruct((NC, N_PAD, D), jnp.float32),
        mesh=mesh,
        scratch_types=[
            pltpu.MemorySpace.VMEM_SHARED((N_PAD, D), jnp.float32),
            pltpu.MemorySpace.VMEM((CPW, CHUNK), jnp.int32),
            pltpu.MemorySpace.VMEM((CPW, CHUNK), jnp.int32),
            pltpu.MemorySpace.VMEM((CHUNK, D), jnp.float32),
            pltpu.SemaphoreType.DMA,
        ],
    )
    def agg_kernel(h_hbm, src_hbm, dst_hbm, out_hbm, acc, src_v, dst_v, rows_v, sem):
        c = lax.axis_index("c")
        s = lax.axis_index("s")
        wid = c * NS + s

        zero = jnp.zeros((16,), jnp.float32)

        def zrow(i, _):
            for j in range(D // 16):
                rows_v[i, pl.ds(j * 16, 16)] = zero
            return 0

        lax.fori_loop(0, CHUNK, zrow, 0)
        for r in range(RPT // CHUNK):
            pltpu.sync_copy(rows_v, acc.at[pl.ds(s * RPT + r * CHUNK, CHUNK)])
        plsc.subcore_barrier()

        pltpu.sync_copy(src_hbm.at[wid], src_v)
        pltpu.sync_copy(dst_hbm.at[wid], dst_v)

        def chunk_body(j, _):
            pltpu.async_copy(h_hbm.at[src_v.at[j]], rows_v, sem).wait()
            pltpu.sync_copy(rows_v, acc.at[dst_v.at[j]], add=True)
            return 0

        lax.fori_loop(0, CPW, chunk_body, 0)
        plsc.subcore_barrier()

        pltpu.sync_copy(acc.at[pl.ds(s * RPT, RPT)],
                        out_hbm.at[c, pl.ds(s * RPT, RPT)])

    return agg_kernel(h, src3, dst3)


def _tc_mlp(x, p0, p1, Wa, ba, g, be, Wb, bb):
    def body(x_ref, p0_ref, p1_ref, wa_ref, ba_ref, g_ref, be_ref, wb_ref,
             bb_ref, o_ref):
        h = x_ref[...] + p0_ref[...] + p1_ref[...]
        t = lax.dot_general(h, wa_ref[...], (((1,), (1,)), ((), ())),
                            preferred_element_type=jnp.float32) + ba_ref[...]
        mu = jnp.mean(t, axis=0, keepdims=True)
        var = jnp.mean((t - mu) * (t - mu), axis=0, keepdims=True)
        t = (t - mu) * lax.rsqrt(var + BN_EPS) * g_ref[...] + be_ref[...]
        t = jnp.maximum(t, 0.0)
        o_ref[...] = lax.dot_general(t, wb_ref[...], (((1,), (1,)), ((), ())),
                                     preferred_element_type=jnp.float32) + bb_ref[...]

    return pl.pallas_call(
        body,
        out_shape=jax.ShapeDtypeStruct((N, D), jnp.float32),
    )(x, p0, p1, Wa, ba.reshape(1, D), g.reshape(1, D), be.reshape(1, D),
      Wb, bb.reshape(1, D))


def _layer(h, src3, dst3, Wa, ba, g, be, Wb, bb):
    p = _sc_aggregate(h, src3, dst3)
    return _tc_mlp(h, p[0, :N], p[1, :N], Wa, ba, g, be, Wb, bb)


def kernel(x, edge_index, W1a, b1a, g1, be1, W1b, b1b,
           W2a, b2a, g2, be2, W2b, b2b):
    src = edge_index[0]
    dst = edge_index[1]
    pad = E_PAD - E
    src3 = jnp.concatenate([src, jnp.zeros((pad,), jnp.int32)]).reshape(NW, CPW, CHUNK)
    dst3 = jnp.concatenate([dst, jnp.full((pad,), N, jnp.int32)]).reshape(NW, CPW, CHUNK)

    h = _layer(x, src3, dst3, W1a, b1a, g1, be1, W1b, b1b)
    h = _layer(h, src3, dst3, W2a, b2a, g2, be2, W2b, b2b)
    return h
